# baseline (speedup 1.0000x reference)
import functools

import jax
import jax.numpy as jnp
from jax.experimental import pallas as pl
from jax.experimental.pallas import tpu as pltpu


def _norm_entries(ei, ew, n):
    src = ei[0]
    dst = ei[1]
    if ew is None:
        ew = jnp.ones(src.shape, jnp.float32)
    ew = ew.astype(jnp.float32)
    is_loop = src == dst
    nz = jnp.where(is_loop, 0.0, ew)
    loop_w = jnp.ones((n,), jnp.float32).at[
        jnp.where(is_loop, src, n)].set(jnp.where(is_loop, ew, 1.0),
                                        mode="drop")
    deg = jnp.zeros((n,), jnp.float32).at[dst].add(nz) + loop_w
    dinv = jnp.where(deg > 0.0, jax.lax.rsqrt(deg), 0.0)

    ar = jnp.arange(n, dtype=src.dtype)
    rows = jnp.concatenate([dst, ar])
    cols = jnp.concatenate([src, ar])
    vals = jnp.concatenate([nz * dinv[dst] * dinv[src],
                            loop_w * dinv * dinv])
    return rows, cols, vals


def _build_adj(entry_list, n, dtype):
    nv = len(entry_list)
    rows = jnp.concatenate([r + v * n for v, (r, _, _) in enumerate(entry_list)])
    cols = jnp.concatenate([c for _, c, _ in entry_list])
    vals = jnp.concatenate([x for _, _, x in entry_list]).astype(dtype)
    slab = jnp.zeros((nv * n, n), dtype)
    return slab.at[rows, cols].add(vals).reshape(nv, n, n)


def _tile(n, target):
    if n % 8:
        return n
    best = 8
    for c in range(8, min(n, target) + 1, 8):
        if n % c == 0:
            best = c
    return best


def _xw_kernel(x_ref, w1_ref, xw_ref):
    xw_ref[...] = jnp.dot(
        x_ref[...].astype(jnp.bfloat16), w1_ref[...],
        preferred_element_type=jnp.float32).astype(jnp.bfloat16)


def _conv1_kernel(a1_ref, xw_ref, b1_ref, w2_ref, feat_ref, z_ref, *, nhid):
    H = nhid
    feats = None
    zcols = []
    for v in range(3):
        h = jnp.dot(a1_ref[v].astype(jnp.bfloat16),
                    xw_ref[:, v * H:(v + 1) * H],
                    preferred_element_type=jnp.float32) + b1_ref[v]
        h = jnp.maximum(h, 0.0)
        feats = h if feats is None else feats + h
        if v != 1:
            k = 0 if v == 0 else 1
            zcols.append(jnp.sum(h * w2_ref[k], axis=1, keepdims=True))
    feat_ref[...] = feats
    z_ref[...] = jnp.concatenate(zcols, axis=1)


def _conv2_kernel(a2_ref, zt_ref, b2c_ref, out_ref):
    s0 = jnp.sum(a2_ref[0].astype(jnp.float32) * zt_ref[0:1, :],
                 axis=1, keepdims=True)
    s1 = jnp.sum(a2_ref[1].astype(jnp.float32) * zt_ref[1:2, :],
                 axis=1, keepdims=True)
    out_ref[...] = s0 + s1 + b2c_ref[0, 0]


def kernel(x, ei1, ei2, ei3, ew1, ew2, ew3,
           w1_v1, b1_v1, w2_v1, b2_v1,
           w1_v2, b1_v2, w2_v2, b2_v2,
           w1_v3, b1_v3, w2_v3, b2_v3, att):
    N, F = x.shape
    H = w1_v1.shape[0]
    adt = jnp.float32

    a1 = _build_adj([_norm_entries(ei1, ew1, N),
                     _norm_entries(ei2, ew2, N),
                     _norm_entries(ei3, ew3, N)], N, adt)
    a2 = _build_adj([_norm_entries(ei1, None, N),
                     _norm_entries(ei3, None, N)], N, adt)

    return a1[0, 0, :] + a2[0, 0, :], jnp.zeros((N, H), x.dtype)

    w1_all = jnp.concatenate(
        [w1_v1.T, w1_v2.T, w1_v3.T], axis=1).astype(jnp.bfloat16)
    b1_all = jnp.stack([b1_v1.reshape(1, H).astype(jnp.float32),
                        b1_v2.reshape(1, H).astype(jnp.float32),
                        b1_v3.reshape(1, H).astype(jnp.float32)])

    s = jax.nn.softmax(att.reshape(3).astype(jnp.float32))
    c0, c2 = s[0], s[1] + s[2]
    w2_rows = jnp.stack([c0 * w2_v1.reshape(1, H).astype(jnp.float32),
                         c2 * w2_v3.reshape(1, H).astype(jnp.float32)])
    b2c = (c0 * b2_v1.reshape(()).astype(jnp.float32)
           + c2 * b2_v3.reshape(()).astype(jnp.float32)).reshape(1, 1)

    TX = _tile(N, 512)
    xw = pl.pallas_call(
        _xw_kernel,
        out_shape=jax.ShapeDtypeStruct((N, 3 * H), jnp.bfloat16),
        grid=(N // TX,),
        in_specs=[pl.BlockSpec((TX, F), lambda i: (i, 0)),
                  pl.BlockSpec((F, 3 * H), lambda i: (0, 0))],
        out_specs=pl.BlockSpec((TX, 3 * H), lambda i: (i, 0)),
        compiler_params=pltpu.CompilerParams(
            dimension_semantics=("parallel",)),
    )(x, w1_all)

    T = _tile(N, 256)
    features, z = pl.pallas_call(
        functools.partial(_conv1_kernel, nhid=H),
        out_shape=(jax.ShapeDtypeStruct((N, H), x.dtype),
                   jax.ShapeDtypeStruct((N, 2), jnp.float32)),
        grid=(N // T,),
        in_specs=[pl.BlockSpec((3, T, N), lambda i: (0, i, 0)),
                  pl.BlockSpec((N, 3 * H), lambda i: (0, 0)),
                  pl.BlockSpec((3, 1, H), lambda i: (0, 0, 0)),
                  pl.BlockSpec((2, 1, H), lambda i: (0, 0, 0))],
        out_specs=[pl.BlockSpec((T, H), lambda i: (i, 0)),
                   pl.BlockSpec((T, 2), lambda i: (i, 0))],
        compiler_params=pltpu.CompilerParams(
            dimension_semantics=("parallel",)),
    )(a1, xw, b1_all, w2_rows)

    zt = z.T
    TC = _tile(N, 256)
    comb = pl.pallas_call(
        _conv2_kernel,
        out_shape=jax.ShapeDtypeStruct((N, 1), x.dtype),
        grid=(N // TC,),
        in_specs=[pl.BlockSpec((2, TC, N), lambda i: (0, i, 0)),
                  pl.BlockSpec((2, N), lambda i: (0, 0)),
                  pl.BlockSpec((1, 1), lambda i: (0, 0))],
        out_specs=pl.BlockSpec((TC, 1), lambda i: (i, 0)),
        compiler_params=pltpu.CompilerParams(
            dimension_semantics=("parallel",)),
    )(a2, zt, b2c)

    return comb.reshape(-1), features


# speedup vs baseline: 3.5057x; 3.5057x over previous
import functools

import jax
import jax.numpy as jnp
from jax.experimental import pallas as pl
from jax.experimental.pallas import tpu as pltpu


def _view_terms(ei, ew, n):
    src = ei[0]
    dst = ei[1]
    if ew is None:
        ew = jnp.ones(src.shape, jnp.float32)
    ew = ew.astype(jnp.float32)
    is_loop = src == dst
    nz = jnp.where(is_loop, 0.0, ew)
    a_raw = jnp.zeros((n, n), jnp.float32).at[dst, src].add(nz)
    loop_w = jnp.ones((n,), jnp.float32).at[
        jnp.where(is_loop, src, n)].set(jnp.where(is_loop, ew, 1.0),
                                        mode="drop")
    deg = jnp.zeros((n,), jnp.float32).at[dst].add(nz) + loop_w
    dinv = jnp.where(deg > 0.0, jax.lax.rsqrt(deg), 0.0)
    return a_raw, loop_w, dinv


def _tile(n, target):
    if n % 8:
        return n
    best = 8
    for c in range(8, min(n, target) + 1, 8):
        if n % c == 0:
            best = c
    return best


def _xw_kernel(x_ref, w1_ref, xw_ref):
    xw_ref[...] = jnp.dot(
        x_ref[...].astype(jnp.bfloat16), w1_ref[...],
        preferred_element_type=jnp.float32).astype(jnp.bfloat16)


def _conv1_kernel(r1_ref, r2_ref, r3_ref, xw_ref, dr_ref, dc_ref, cf_ref,
                  b1_ref, w2_ref, feat_ref, z_ref, *, nhid, tile):
    H, T = nhid, tile
    i = pl.program_id(0)
    feats = None
    zcols = []
    for v, r_ref in enumerate((r1_ref, r2_ref, r3_ref)):
        a = (r_ref[...] * dr_ref[v]).astype(jnp.bfloat16)
        xw_v = xw_ref[:, v * H:(v + 1) * H]
        m = jnp.dot(a, xw_v, preferred_element_type=jnp.float32)
        xw_t = xw_ref[pl.ds(i * T, T),
                      v * H:(v + 1) * H].astype(jnp.float32)
        s = m + cf_ref[v] * xw_t
        h = jnp.maximum(dc_ref[v] * s + b1_ref[v], 0.0)
        feats = h if feats is None else feats + h
        if v != 1:
            k = 0 if v == 0 else 1
            zcols.append(jnp.sum(h * w2_ref[k], axis=1, keepdims=True))
    feat_ref[...] = feats
    z_ref[...] = jnp.concatenate(zcols, axis=1)


def _conv2_kernel(q1_ref, q3_ref, y_ref, dc_ref, dg_ref, b2c_ref, out_ref):
    s0 = jnp.sum(q1_ref[...] * y_ref[0:1, :], axis=1, keepdims=True)
    s1 = jnp.sum(q3_ref[...] * y_ref[1:2, :], axis=1, keepdims=True)
    out_ref[...] = dc_ref[:, 0:1] * s0 + dc_ref[:, 1:2] * s1 \
        + dg_ref[...] + b2c_ref[0, 0]


def kernel(x, ei1, ei2, ei3, ew1, ew2, ew3,
           w1_v1, b1_v1, w2_v1, b2_v1,
           w1_v2, b1_v2, w2_v2, b2_v2,
           w1_v3, b1_v3, w2_v3, b2_v3, att):
    N, F = x.shape
    H = w1_v1.shape[0]

    r1, lw1, di1 = _view_terms(ei1, ew1, N)
    r2, lw2, di2 = _view_terms(ei2, ew2, N)
    r3, lw3, di3 = _view_terms(ei3, ew3, N)
    q1, ql1, qd1 = _view_terms(ei1, None, N)
    q3, ql3, qd3 = _view_terms(ei3, None, N)

    drow = jnp.stack([di1.reshape(1, N), di2.reshape(1, N),
                      di3.reshape(1, N)])
    dcol = jnp.stack([di1.reshape(N, 1), di2.reshape(N, 1),
                      di3.reshape(N, 1)])
    coef = jnp.stack([(lw1 * di1).reshape(N, 1), (lw2 * di2).reshape(N, 1),
                      (lw3 * di3).reshape(N, 1)])

    w1_all = jnp.concatenate(
        [w1_v1.T, w1_v2.T, w1_v3.T], axis=1).astype(jnp.bfloat16)
    b1_all = jnp.stack([b1_v1.reshape(1, H).astype(jnp.float32),
                        b1_v2.reshape(1, H).astype(jnp.float32),
                        b1_v3.reshape(1, H).astype(jnp.float32)])

    s = jax.nn.softmax(att.reshape(3).astype(jnp.float32))
    c0, c2 = s[0], s[1] + s[2]
    w2_rows = jnp.stack([c0 * w2_v1.reshape(1, H).astype(jnp.float32),
                         c2 * w2_v3.reshape(1, H).astype(jnp.float32)])
    b2c = (c0 * b2_v1.reshape(()).astype(jnp.float32)
           + c2 * b2_v3.reshape(()).astype(jnp.float32)).reshape(1, 1)

    TX = _tile(N, 512)
    xw = pl.pallas_call(
        _xw_kernel,
        out_shape=jax.ShapeDtypeStruct((N, 3 * H), jnp.bfloat16),
        grid=(N // TX,),
        in_specs=[pl.BlockSpec((TX, F), lambda i: (i, 0)),
                  pl.BlockSpec((F, 3 * H), lambda i: (0, 0))],
        out_specs=pl.BlockSpec((TX, 3 * H), lambda i: (i, 0)),
        compiler_params=pltpu.CompilerParams(
            dimension_semantics=("parallel",)),
    )(x, w1_all)

    T = _tile(N, 256)
    features, z = pl.pallas_call(
        functools.partial(_conv1_kernel, nhid=H, tile=T),
        out_shape=(jax.ShapeDtypeStruct((N, H), x.dtype),
                   jax.ShapeDtypeStruct((N, 2), jnp.float32)),
        grid=(N // T,),
        in_specs=[pl.BlockSpec((T, N), lambda i: (i, 0)),
                  pl.BlockSpec((T, N), lambda i: (i, 0)),
                  pl.BlockSpec((T, N), lambda i: (i, 0)),
                  pl.BlockSpec((N, 3 * H), lambda i: (0, 0)),
                  pl.BlockSpec((3, 1, N), lambda i: (0, 0, 0)),
                  pl.BlockSpec((3, T, 1), lambda i: (0, i, 0)),
                  pl.BlockSpec((3, T, 1), lambda i: (0, i, 0)),
                  pl.BlockSpec((3, 1, H), lambda i: (0, 0, 0)),
                  pl.BlockSpec((2, 1, H), lambda i: (0, 0, 0))],
        out_specs=[pl.BlockSpec((T, H), lambda i: (i, 0)),
                   pl.BlockSpec((T, 2), lambda i: (i, 0))],
        compiler_params=pltpu.CompilerParams(
            dimension_semantics=("parallel",)),
    )(r1, r2, r3, xw, drow, dcol, coef, b1_all, w2_rows)

    y = jnp.stack([qd1 * z[:, 0], qd3 * z[:, 1]])
    dg = (qd1 * ql1 * y[0] + qd3 * ql3 * y[1]).reshape(N, 1)
    dc2 = jnp.stack([qd1, qd3], axis=1)
    TC = _tile(N, 256)
    comb = pl.pallas_call(
        _conv2_kernel,
        out_shape=jax.ShapeDtypeStruct((N, 1), x.dtype),
        grid=(N // TC,),
        in_specs=[pl.BlockSpec((TC, N), lambda i: (i, 0)),
                  pl.BlockSpec((TC, N), lambda i: (i, 0)),
                  pl.BlockSpec((2, N), lambda i: (0, 0)),
                  pl.BlockSpec((TC, 2), lambda i: (i, 0)),
                  pl.BlockSpec((TC, 1), lambda i: (i, 0)),
                  pl.BlockSpec((1, 1), lambda i: (0, 0))],
        out_specs=pl.BlockSpec((TC, 1), lambda i: (i, 0)),
        compiler_params=pltpu.CompilerParams(
            dimension_semantics=("parallel",)),
    )(q1, q3, y, dc2, dg, b2c)

    return comb.reshape(-1), features
